# Initial kernel scaffold; baseline (speedup 1.0000x reference)
import jax
import jax.numpy as jnp
from jax.experimental import pallas as pl


def kernel(num, cat_indices, tables, gamma, beta, W, b):
    raise NotImplementedError("write your pallas kernel here")


# speedup vs baseline: 1.1470x; 1.1470x over previous
import functools

import jax
import jax.numpy as jnp
from jax import lax
from jax.experimental import pallas as pl
from jax.experimental.pallas import tpu as pltpu
from jax.experimental.pallas import tpu_sc as plsc

B = 16384
N_CAT = 26
N_NUM = 13
VOCAB = 100000
EMB = 16
OUT = 128

NC = 2
NS = 16
NW = NC * NS
LOOKUPS = B * N_CAT
PER_W = LOOKUPS // NW
IDX_ROWS = PER_W // 128
CHUNK_ROWS = 13
N_CHUNKS = IDX_ROWS // CHUNK_ROWS
CHUNK = CHUNK_ROWS * 128


def _sc_gather(tab_flat, idx2d):
    mesh = plsc.VectorSubcoreMesh(core_axis_name="c", subcore_axis_name="s")

    @functools.partial(
        pl.kernel,
        mesh=mesh,
        out_type=jax.ShapeDtypeStruct((LOOKUPS, EMB), jnp.float32),
        scratch_types=[
            pltpu.VMEM((IDX_ROWS, 128), jnp.int32),
            pltpu.VMEM((CHUNK, EMB), jnp.float32),
            pltpu.SemaphoreType.DMA,
        ],
        compiler_params=pltpu.CompilerParams(use_tc_tiling_on_sc=False),
    )
    def k(tab_hbm, idx_hbm, out_hbm, idx_v, rows_v, sem):
        wid = lax.axis_index("s") * NC + lax.axis_index("c")
        row0 = wid * IDX_ROWS
        pltpu.sync_copy(idx_hbm.at[pl.ds(row0, IDX_ROWS)], idx_v)

        base = wid * PER_W
        lane = lax.iota(jnp.int32, 16)

        def add_offsets(r, carry):
            for lb in range(8):
                pos0 = base + r * 128 + lb * 16
                s0 = lax.rem(pos0, N_CAT)
                t = s0 + lane
                f = jnp.where(t >= N_CAT, t - N_CAT, t)
                sl = (r, pl.ds(lb * 16, 16))
                idx_v[sl] = idx_v[sl] + f * VOCAB
            return carry

        lax.fori_loop(0, IDX_ROWS, add_offsets, 0)

        for c in range(N_CHUNKS):
            copies = []
            for j in range(CHUNK_ROWS):
                copies.append(pltpu.async_copy(
                    tab_hbm.at[idx_v.at[c * CHUNK_ROWS + j]],
                    rows_v.at[pl.ds(j * 128, 128)],
                    sem,
                ))
            for cp in copies:
                cp.wait()
            pltpu.sync_copy(
                rows_v,
                out_hbm.at[pl.ds(wid * PER_W + c * CHUNK, CHUNK)],
            )

    return k(tab_flat, idx2d)


IN_E = N_CAT * EMB
BLK = 2048
G = B // BLK


def _tc_tail_body(num_ref, emb_ref, gn, ge, bn, be, w1, w2, bb,
                  out_ref, sn, sqn, se, sqe):
    p = pl.program_id(0)
    i = pl.program_id(1)

    @pl.when(p == 0)
    def _stats():
        nblk = num_ref[...]
        eblk = emb_ref[...]
        s1 = jnp.sum(nblk, axis=0, keepdims=True)
        q1 = jnp.sum(nblk * nblk, axis=0, keepdims=True)
        s2 = jnp.sum(eblk, axis=0, keepdims=True)
        q2 = jnp.sum(eblk * eblk, axis=0, keepdims=True)

        @pl.when(i == 0)
        def _():
            sn[...] = s1
            sqn[...] = q1
            se[...] = s2
            sqe[...] = q2

        @pl.when(i > 0)
        def _():
            sn[...] += s1
            sqn[...] += q1
            se[...] += s2
            sqe[...] += q2

        @pl.when(i == G - 1)
        def _():
            inv_b = 1.0 / B
            mn = sn[...] * inv_b
            vn = sqn[...] * inv_b - mn * mn
            scale_n = gn[...] * lax.rsqrt(vn + 1e-5)
            sn[...] = scale_n
            sqn[...] = bn[...] - mn * scale_n
            me = se[...] * inv_b
            ve = sqe[...] * inv_b - me * me
            scale_e = ge[...] * lax.rsqrt(ve + 1e-5)
            se[...] = scale_e
            sqe[...] = be[...] - me * scale_e

    @pl.when(p == 1)
    def _matmul():
        h_n = jnp.maximum(num_ref[...] * sn[...] + sqn[...], 0.0)
        h_e = jnp.maximum(emb_ref[...] * se[...] + sqe[...], 0.0)
        dn = (((1,), (1,)), ((), ()))
        out_ref[...] = (
            lax.dot_general(h_n, w1[...], dn,
                            preferred_element_type=jnp.float32,
                            precision=lax.Precision.HIGHEST)
            + lax.dot_general(h_e, w2[...], dn,
                              preferred_element_type=jnp.float32,
                              precision=lax.Precision.HIGHEST)
            + bb[...]
        )


def _tc_tail(num, emb, gn, ge, bn, be, w1, w2, bb):
    full = lambda shape: pl.BlockSpec(shape, lambda p, i: (0, 0))
    blk = lambda shape: pl.BlockSpec(shape, lambda p, i: (i, 0))
    return pl.pallas_call(
        _tc_tail_body,
        grid=(2, G),
        in_specs=[
            blk((BLK, N_NUM)),
            blk((BLK, IN_E)),
            full((1, N_NUM)),
            full((1, IN_E)),
            full((1, N_NUM)),
            full((1, IN_E)),
            full((OUT, N_NUM)),
            full((OUT, IN_E)),
            full((1, OUT)),
        ],
        out_specs=blk((BLK, OUT)),
        out_shape=jax.ShapeDtypeStruct((B, OUT), jnp.float32),
        scratch_shapes=[
            pltpu.VMEM((1, N_NUM), jnp.float32),
            pltpu.VMEM((1, N_NUM), jnp.float32),
            pltpu.VMEM((1, IN_E), jnp.float32),
            pltpu.VMEM((1, IN_E), jnp.float32),
        ],
    )(num, emb, gn, ge, bn, be, w1, w2, bb)


def kernel(num, cat_indices, tables, gamma, beta, W, b):
    tab_flat = tables.reshape(N_CAT * VOCAB, EMB)
    idx2d = cat_indices.reshape(LOOKUPS // 128, 128)
    emb_flat = _sc_gather(tab_flat, idx2d)
    emb = emb_flat.reshape(B, IN_E)
    out = _tc_tail(
        num, emb,
        gamma[:N_NUM].reshape(1, N_NUM), gamma[N_NUM:].reshape(1, IN_E),
        beta[:N_NUM].reshape(1, N_NUM), beta[N_NUM:].reshape(1, IN_E),
        W[:, :N_NUM], W[:, N_NUM:],
        b.reshape(1, OUT),
    )
    return out
